# Initial kernel scaffold; baseline (speedup 1.0000x reference)
import functools

import jax
import jax.numpy as jnp
from jax import lax
from jax.experimental import pallas as pl
from jax.experimental.pallas import tpu as pltpu
from jax.experimental.pallas import tpu_sc as plsc

INF = jnp.int32(2**31 - 1)
NC = 2
NS = 16
ROWS = 1024
CHUNK = 128


def _k1_body(feats_ref, wc4_ref, wm_ref, bm_ref, z0_ref, z1_ref, z2_ref,
             z3_ref, init_ref):
    h = feats_ref[...]
    z = jnp.dot(h, wc4_ref[...], preferred_element_type=jnp.float32)
    c = h.shape[1]
    z0_ref[...] = z[:, 0 * c:1 * c]
    z1_ref[...] = z[:, 1 * c:2 * c]
    z2_ref[...] = z[:, 2 * c:3 * c]
    z3_ref[...] = z[:, 3 * c:4 * c]
    s = jnp.sum(h * wm_ref[...], axis=1, keepdims=True) + bm_ref[...]
    init_ref[...] = jax.nn.sigmoid(s)


def _k1(featsN, Wc4, wm2, bm2):
    n, c = featsN.shape
    grid = (n // ROWS,)
    zspec = pl.BlockSpec((ROWS, c), lambda i: (i, 0))
    return pl.pallas_call(
        _k1_body,
        grid=grid,
        in_specs=[
            pl.BlockSpec((ROWS, c), lambda i: (i, 0)),
            pl.BlockSpec((c, 4 * c), lambda i: (0, 0)),
            pl.BlockSpec((1, c), lambda i: (0, 0)),
            pl.BlockSpec((1, 1), lambda i: (0, 0)),
        ],
        out_specs=[zspec, zspec, zspec, zspec,
                   pl.BlockSpec((ROWS, 1), lambda i: (i, 0))],
        out_shape=[jax.ShapeDtypeStruct((n, c), jnp.float32)] * 4
        + [jax.ShapeDtypeStruct((n, 1), jnp.float32)],
    )(featsN, Wc4, wm2, bm2)


def _k2_body(z1, z2, z3, a0, a1, a2, g1, g2, g3,
             i0, i1, i2, r1, r2, r3, s1, s2, s3):
    wid = lax.axis_index("s") * NC + lax.axis_index("c")
    n = z1.shape[0]
    per_w = n // (NC * NS)
    for j in range(per_w // CHUNK):
        start = wid * per_w + j * CHUNK
        pltpu.sync_copy(a0.at[pl.ds(start, CHUNK)], i0)
        pltpu.sync_copy(a1.at[pl.ds(start, CHUNK)], i1)
        pltpu.sync_copy(a2.at[pl.ds(start, CHUNK)], i2)
        c1 = pltpu.async_copy(z1.at[i0], r1, s1)
        c2 = pltpu.async_copy(z2.at[i1], r2, s2)
        c3 = pltpu.async_copy(z3.at[i2], r3, s3)
        c1.wait()
        c2.wait()
        c3.wait()
        pltpu.sync_copy(r1, g1.at[pl.ds(start, CHUNK)])
        pltpu.sync_copy(r2, g2.at[pl.ds(start, CHUNK)])
        pltpu.sync_copy(r3, g3.at[pl.ds(start, CHUNK)])


def _k2(z1, z2, z3, a0f, a1f, a2f):
    n, c = z1.shape
    mesh = plsc.VectorSubcoreMesh(core_axis_name="c", subcore_axis_name="s")
    out = jax.ShapeDtypeStruct((n, c), jnp.float32)
    run = functools.partial(
        pl.kernel,
        out_type=[out, out, out],
        mesh=mesh,
        scratch_types=[pltpu.VMEM((CHUNK,), jnp.int32)] * 3
        + [pltpu.VMEM((CHUNK, c), jnp.float32)] * 3
        + [pltpu.SemaphoreType.DMA] * 3,
    )(_k2_body)
    return run(z1, z2, z3, a0f, a1f, a2f)


def _k3_body(z0_ref, g1_ref, g2_ref, g3_ref, bc_ref, wm_ref, bm_ref,
             bf_ref, pred_ref):
    h = z0_ref[...] + g1_ref[...] + g2_ref[...] + g3_ref[...] + bc_ref[...]
    mu = jnp.mean(h, axis=1, keepdims=True)
    d = h - mu
    var = jnp.mean(d * d, axis=1, keepdims=True)
    bf = d * lax.rsqrt(var + 1e-5)
    bf_ref[...] = bf
    p = jnp.sum(bf * wm_ref[...], axis=1, keepdims=True) + bm_ref[...]
    pred_ref[...] = jax.nn.sigmoid(p)


def _k3(z0, g1, g2, g3, bc2, wm2, bm2):
    n, c = z0.shape
    grid = (n // ROWS,)
    zspec = pl.BlockSpec((ROWS, c), lambda i: (i, 0))
    return pl.pallas_call(
        _k3_body,
        grid=grid,
        in_specs=[zspec, zspec, zspec, zspec,
                  pl.BlockSpec((1, c), lambda i: (0, 0)),
                  pl.BlockSpec((1, c), lambda i: (0, 0)),
                  pl.BlockSpec((1, 1), lambda i: (0, 0))],
        out_specs=[zspec, pl.BlockSpec((ROWS, 1), lambda i: (i, 0))],
        out_shape=[jax.ShapeDtypeStruct((n, c), jnp.float32),
                   jax.ShapeDtypeStruct((n, 1), jnp.float32)],
    )(z0, g1, g2, g3, bc2, wm2, bm2)


def _k4_body(a0h, a1h, a2h, prh, inh, qih, dih, tgh, sch, dph,
             A0, A1, A2, PR, SCR, DQ, QU, TG):
    bn = a0h.shape[0]
    fpad = a0h.shape[1]
    wid = lax.axis_index("s") * NC + lax.axis_index("c")

    @pl.when(wid < bn)
    def _():
        b = wid
        pltpu.sync_copy(a0h.at[b], A0)
        pltpu.sync_copy(a1h.at[b], A1)
        pltpu.sync_copy(a2h.at[b], A2)
        pltpu.sync_copy(prh.at[b], PR)
        pltpu.sync_copy(inh.at[b], SCR)
        pltpu.sync_copy(qih.at[b], QU.at[pl.ds(0, fpad)])
        pltpu.sync_copy(dih.at[b], DQ)
        pltpu.sync_copy(tgh, TG)
        iota16 = lax.iota(jnp.int32, 16)

        def outer_body(carry):
            lo, tail, level, token = carry
            hi = tail

            def inner_cond(c):
                return c[0] < hi

            def inner_body(c):
                base, tl, tok = c
                fv = QU[pl.ds(base, 16)]
                m = (base + iota16) < hi
                f = jnp.where(m, fv, 0)
                a0v = plsc.load_gather(A0, [f])
                a1v = plsc.load_gather(A1, [f])
                a2v = plsc.load_gather(A2, [f])
                d0 = plsc.load_gather(DQ, [a0v])
                d1 = plsc.load_gather(DQ, [a1v])
                d2 = plsc.load_gather(DQ, [a2v])
                s0 = plsc.load_gather(SCR, [a0v])
                s1 = plsc.load_gather(SCR, [a1v])
                s2 = plsc.load_gather(SCR, [a2v])
                v0 = d0 < level
                v1 = d1 < level
                v2 = d2 < level
                neg = jnp.float32(-1e30)
                nb = jnp.maximum(jnp.maximum(jnp.where(v0, s0, neg),
                                             jnp.where(v1, s1, neg)),
                                 jnp.where(v2, s2, neg))
                has = v0 | v1 | v2
                nbv = jnp.where(has, nb, jnp.float32(1.0))
                pf = plsc.load_gather(PR, [f])
                sf = plsc.load_gather(SCR, [f])
                val = jnp.minimum(jnp.maximum(pf, sf), nbv)
                plsc.store_scatter(SCR, [f], val, mask=m)
                dnew = jnp.zeros((16,), jnp.int32) + (level + 1)

                def expand(av, tl, tok):
                    dd = plsc.load_gather(DQ, [av])
                    cand = m & (dd == INF)
                    tokv = tok + iota16
                    plsc.store_scatter(TG, [av], tokv, mask=cand)
                    tt = plsc.load_gather(TG, [av])
                    win = cand & (tt == tokv)
                    plsc.store_scatter(DQ, [av], dnew, mask=cand)
                    wi = win.astype(jnp.int32)
                    pos = plsc.cumsum(wi) + (tl - 1)
                    plsc.store_scatter(QU, [pos], av, mask=win)
                    return tl + jnp.sum(wi), tok + 16

                tl, tok = expand(a0v, tl, tok)
                tl, tok = expand(a1v, tl, tok)
                tl, tok = expand(a2v, tl, tok)
                return base + 16, tl, tok

            _, tail, token = lax.while_loop(inner_cond, inner_body,
                                            (lo, tail, token))
            return hi, tail, level + jnp.int32(1), token

        def outer_cond(carry):
            return carry[0] < carry[1]

        lax.while_loop(outer_cond, outer_body,
                       (jnp.int32(0), jnp.int32(1), jnp.int32(0),
                        jnp.int32(0)))
        pltpu.sync_copy(SCR, sch.at[b])
        pltpu.sync_copy(DQ, dph.at[b])


def _k4(a0l, a1l, a2l, pred2, init2, qi, di, tgi):
    bn, fpad = a0l.shape
    mesh = plsc.VectorSubcoreMesh(core_axis_name="c", subcore_axis_name="s")
    run = functools.partial(
        pl.kernel,
        out_type=[jax.ShapeDtypeStruct((bn, fpad), jnp.float32),
                  jax.ShapeDtypeStruct((bn, fpad), jnp.int32)],
        mesh=mesh,
        scratch_types=[pltpu.VMEM((fpad,), jnp.int32)] * 3
        + [pltpu.VMEM((fpad,), jnp.float32)] * 2
        + [pltpu.VMEM((fpad,), jnp.int32),
           pltpu.VMEM((fpad + 16,), jnp.int32),
           pltpu.VMEM((fpad,), jnp.int32)],
    )(_k4_body)
    return run(a0l, a1l, a2l, pred2, init2, qi, di, tgi)


def _k5_body(feats_ref, bf_ref, depth_ref, out_ref):
    reached = depth_ref[...] != INF
    out_ref[...] = jnp.where(reached, bf_ref[...], feats_ref[...])


def _k5(featsN, bf, depthN):
    n, c = featsN.shape
    grid = (n // ROWS,)
    zspec = pl.BlockSpec((ROWS, c), lambda i: (i, 0))
    return pl.pallas_call(
        _k5_body,
        grid=grid,
        in_specs=[zspec, zspec, pl.BlockSpec((ROWS, 1), lambda i: (i, 0))],
        out_specs=zspec,
        out_shape=jax.ShapeDtypeStruct((n, c), jnp.float32),
    )(featsN, bf, depthN)


def kernel(x, face_adj, anchors, Wc, bc, Wm, bm):
    bn, cn, fn = x.shape
    fpad = ((fn + ROWS - 1) // ROWS) * ROWS
    while (bn * fpad) % (NC * NS * CHUNK) != 0:
        fpad += ROWS
    n = bn * fpad

    feats = jnp.transpose(x, (0, 2, 1))
    featsP = jnp.pad(feats, ((0, 0), (0, fpad - fn), (0, 0)))
    featsN = featsP.reshape(n, cn)

    Wc4 = jnp.concatenate([Wc[k * cn:(k + 1) * cn] for k in range(4)],
                          axis=1)
    wm2 = Wm.reshape(1, cn)
    bm2 = bm.reshape(1, 1)
    bc2 = bc.reshape(1, cn)

    adjP = jnp.pad(face_adj, ((0, 0), (0, fpad - fn), (0, 0)))
    a0l = adjP[:, :, 0]
    a1l = adjP[:, :, 1]
    a2l = adjP[:, :, 2]
    offs = (jnp.arange(bn, dtype=jnp.int32) * fpad)[:, None]
    a0f = (a0l + offs).reshape(n)
    a1f = (a1l + offs).reshape(n)
    a2f = (a2l + offs).reshape(n)

    qi = jnp.zeros((bn, fpad), jnp.int32).at[:, 0].set(anchors)
    di = jnp.full((bn, fpad), INF, jnp.int32).at[
        jnp.arange(bn), anchors].set(-1)
    tgi = jnp.full((fpad,), -1, jnp.int32)

    z0, z1, z2, z3, init = _k1(featsN, Wc4, wm2, bm2)
    g1, g2, g3 = _k2(z1, z2, z3, a0f, a1f, a2f)
    bf, pred = _k3(z0, g1, g2, g3, bc2, wm2, bm2)
    scores, depth = _k4(a0l, a1l, a2l, pred.reshape(bn, fpad),
                        init.reshape(bn, fpad), qi, di, tgi)
    outfeat = _k5(featsN, bf, depth.reshape(n, 1))

    final_features = outfeat.reshape(bn, fpad, cn)[:, :fn, :]
    final_scores = scores[:, :fn].reshape(bn, fn, 1)
    return final_features, final_scores


# speedup vs baseline: 45.6378x; 45.6378x over previous
import functools

import jax
import jax.numpy as jnp
from jax import lax
from jax.experimental import pallas as pl
from jax.experimental.pallas import tpu as pltpu
from jax.experimental.pallas import tpu_sc as plsc

INF = 2**31 - 1
NC = 2
NS = 16
ROWS = 1024
CHUNK = 128


def _k1_body(feats_ref, wc4_ref, wm_ref, bm_ref, z0_ref, z1_ref, z2_ref,
             z3_ref, init_ref):
    h = feats_ref[...]
    z = jnp.dot(h, wc4_ref[...], preferred_element_type=jnp.float32)
    c = h.shape[1]
    z0_ref[...] = z[:, 0 * c:1 * c]
    z1_ref[...] = z[:, 1 * c:2 * c]
    z2_ref[...] = z[:, 2 * c:3 * c]
    z3_ref[...] = z[:, 3 * c:4 * c]
    s = jnp.sum(h * wm_ref[...], axis=1, keepdims=True) + bm_ref[...]
    init_ref[...] = jax.nn.sigmoid(s)


def _k1(featsN, Wc4, wm2, bm2):
    n, c = featsN.shape
    grid = (n // ROWS,)
    zspec = pl.BlockSpec((ROWS, c), lambda i: (i, 0))
    return pl.pallas_call(
        _k1_body,
        grid=grid,
        in_specs=[
            pl.BlockSpec((ROWS, c), lambda i: (i, 0)),
            pl.BlockSpec((c, 4 * c), lambda i: (0, 0)),
            pl.BlockSpec((1, c), lambda i: (0, 0)),
            pl.BlockSpec((1, 1), lambda i: (0, 0)),
        ],
        out_specs=[zspec, zspec, zspec, zspec,
                   pl.BlockSpec((ROWS, 1), lambda i: (i, 0))],
        out_shape=[jax.ShapeDtypeStruct((n, c), jnp.float32)] * 4
        + [jax.ShapeDtypeStruct((n, 1), jnp.float32)],
    )(featsN, Wc4, wm2, bm2)


def _k2_body(z1, z2, z3, a0, a1, a2, g1, g2, g3,
             i0, i1, i2, r1, r2, r3, s1, s2, s3):
    wid = lax.axis_index("s") * NC + lax.axis_index("c")
    n = z1.shape[0]
    per_w = n // (NC * NS)
    for j in range(per_w // CHUNK):
        start = wid * per_w + j * CHUNK
        pltpu.sync_copy(a0.at[pl.ds(start, CHUNK)], i0)
        pltpu.sync_copy(a1.at[pl.ds(start, CHUNK)], i1)
        pltpu.sync_copy(a2.at[pl.ds(start, CHUNK)], i2)
        c1 = pltpu.async_copy(z1.at[i0], r1, s1)
        c2 = pltpu.async_copy(z2.at[i1], r2, s2)
        c3 = pltpu.async_copy(z3.at[i2], r3, s3)
        c1.wait()
        c2.wait()
        c3.wait()
        pltpu.sync_copy(r1, g1.at[pl.ds(start, CHUNK)])
        pltpu.sync_copy(r2, g2.at[pl.ds(start, CHUNK)])
        pltpu.sync_copy(r3, g3.at[pl.ds(start, CHUNK)])


def _k2(z1, z2, z3, a0f, a1f, a2f):
    n, c = z1.shape
    mesh = plsc.VectorSubcoreMesh(core_axis_name="c", subcore_axis_name="s")
    out = jax.ShapeDtypeStruct((n, c), jnp.float32)
    run = functools.partial(
        pl.kernel,
        out_type=[out, out, out],
        mesh=mesh,
        scratch_types=[pltpu.VMEM((CHUNK,), jnp.int32)] * 3
        + [pltpu.VMEM((CHUNK, c), jnp.float32)] * 3
        + [pltpu.SemaphoreType.DMA] * 3,
    )(_k2_body)
    return run(z1, z2, z3, a0f, a1f, a2f)


def _k3_body(z0_ref, g1_ref, g2_ref, g3_ref, bc_ref, wm_ref, bm_ref,
             bf_ref, pred_ref):
    h = z0_ref[...] + g1_ref[...] + g2_ref[...] + g3_ref[...] + bc_ref[...]
    mu = jnp.mean(h, axis=1, keepdims=True)
    d = h - mu
    var = jnp.mean(d * d, axis=1, keepdims=True)
    bf = d * lax.rsqrt(var + 1e-5)
    bf_ref[...] = bf
    p = jnp.sum(bf * wm_ref[...], axis=1, keepdims=True) + bm_ref[...]
    pred_ref[...] = jax.nn.sigmoid(p)


def _k3(z0, g1, g2, g3, bc2, wm2, bm2):
    n, c = z0.shape
    grid = (n // ROWS,)
    zspec = pl.BlockSpec((ROWS, c), lambda i: (i, 0))
    return pl.pallas_call(
        _k3_body,
        grid=grid,
        in_specs=[zspec, zspec, zspec, zspec,
                  pl.BlockSpec((1, c), lambda i: (0, 0)),
                  pl.BlockSpec((1, c), lambda i: (0, 0)),
                  pl.BlockSpec((1, 1), lambda i: (0, 0))],
        out_specs=[zspec, pl.BlockSpec((ROWS, 1), lambda i: (i, 0))],
        out_shape=[jax.ShapeDtypeStruct((n, c), jnp.float32),
                   jax.ShapeDtypeStruct((n, 1), jnp.float32)],
    )(z0, g1, g2, g3, bc2, wm2, bm2)


def _k4_body(a0h, a1h, a2h, prh, inh, qih, dih, tgh, sch, dph,
             A0, A1, A2, PR, SCR, DQ, QU, TG):
    bn = a0h.shape[0]
    fpad = a0h.shape[1]
    wid = lax.axis_index("s") * NC + lax.axis_index("c")
    if True:
        b = jnp.minimum(wid, bn - 1)
        pltpu.sync_copy(a0h.at[b], A0)
        pltpu.sync_copy(a1h.at[b], A1)
        pltpu.sync_copy(a2h.at[b], A2)
        pltpu.sync_copy(prh.at[b], PR)
        pltpu.sync_copy(inh.at[b], SCR)
        pltpu.sync_copy(qih.at[b], QU.at[pl.ds(0, fpad)])
        pltpu.sync_copy(dih.at[b], DQ)
        pltpu.sync_copy(tgh, TG)
        iota16 = lax.iota(jnp.int32, 16)

        def step(carry):
            base, hi, tl, level, tok = carry
            new_lvl = base >= hi
            level = jnp.where(new_lvl, level + 1, level)
            base = jnp.where(new_lvl, hi, base)
            hi = jnp.where(new_lvl, tl, hi)
            lanes = base + iota16
            m = lanes < hi
            fv = plsc.load_gather(QU, [jnp.where(m, lanes, 0)])
            f = jnp.where(m, fv, 0)
            a0v = plsc.load_gather(A0, [f])
            a1v = plsc.load_gather(A1, [f])
            a2v = plsc.load_gather(A2, [f])
            d0 = plsc.load_gather(DQ, [a0v])
            d1 = plsc.load_gather(DQ, [a1v])
            d2 = plsc.load_gather(DQ, [a2v])
            s0 = plsc.load_gather(SCR, [a0v])
            s1 = plsc.load_gather(SCR, [a1v])
            s2 = plsc.load_gather(SCR, [a2v])
            v0 = d0 < level
            v1 = d1 < level
            v2 = d2 < level
            neg = jnp.float32(-1e30)
            nb = jnp.maximum(jnp.maximum(jnp.where(v0, s0, neg),
                                         jnp.where(v1, s1, neg)),
                             jnp.where(v2, s2, neg))
            has = v0 | v1 | v2
            nbv = jnp.where(has, nb, jnp.float32(1.0))
            pf = plsc.load_gather(PR, [f])
            sf = plsc.load_gather(SCR, [f])
            val = jnp.minimum(jnp.maximum(pf, sf), nbv)
            plsc.store_scatter(SCR, [f], val, mask=m)
            dnew = jnp.zeros((16,), jnp.int32) + (level + 1)

            def expand(av, tl, tok):
                dd = plsc.load_gather(DQ, [av])
                cand = m & (dd == INF)
                tokv = tok + iota16
                plsc.store_scatter(TG, [av], tokv, mask=cand)
                tt = plsc.load_gather(TG, [av])
                win = cand & (tt == tokv)
                plsc.store_scatter(DQ, [av], dnew, mask=cand)
                wi = win.astype(jnp.int32)
                pos = plsc.cumsum(wi) + (tl - 1)
                plsc.store_scatter(QU, [pos], av, mask=win)
                return tl + jnp.sum(wi), tok + 16

            tl, tok = expand(a0v, tl, tok)
            tl, tok = expand(a1v, tl, tok)
            tl, tok = expand(a2v, tl, tok)
            return base + 16, hi, tl, level, tok

        def not_done(carry):
            base, hi, tl, _, _ = carry
            return (base < hi) | (hi < tl)

        lax.while_loop(not_done, step,
                       (jnp.int32(0), jnp.int32(1), jnp.int32(1),
                        jnp.int32(0), jnp.int32(0)))

        @pl.when(wid < bn)
        def _():
            pltpu.sync_copy(SCR, sch.at[b])
            pltpu.sync_copy(DQ, dph.at[b])


def _k4(a0l, a1l, a2l, pred2, init2, qi, di, tgi):
    bn, fpad = a0l.shape
    mesh = plsc.VectorSubcoreMesh(core_axis_name="c", subcore_axis_name="s")
    run = functools.partial(
        pl.kernel,
        out_type=[jax.ShapeDtypeStruct((bn, fpad), jnp.float32),
                  jax.ShapeDtypeStruct((bn, fpad), jnp.int32)],
        mesh=mesh,
        scratch_types=[pltpu.VMEM((fpad,), jnp.int32)] * 3
        + [pltpu.VMEM((fpad,), jnp.float32)] * 2
        + [pltpu.VMEM((fpad,), jnp.int32),
           pltpu.VMEM((fpad + 16,), jnp.int32),
           pltpu.VMEM((fpad,), jnp.int32)],
        compiler_params=pltpu.CompilerParams(needs_layout_passes=False),
    )(_k4_body)
    return run(a0l, a1l, a2l, pred2, init2, qi, di, tgi)


def _k5_body(feats_ref, bf_ref, depth_ref, out_ref):
    reached = depth_ref[...] != INF
    out_ref[...] = jnp.where(reached, bf_ref[...], feats_ref[...])


def _k5(featsN, bf, depthN):
    n, c = featsN.shape
    grid = (n // ROWS,)
    zspec = pl.BlockSpec((ROWS, c), lambda i: (i, 0))
    return pl.pallas_call(
        _k5_body,
        grid=grid,
        in_specs=[zspec, zspec, pl.BlockSpec((ROWS, 1), lambda i: (i, 0))],
        out_specs=zspec,
        out_shape=jax.ShapeDtypeStruct((n, c), jnp.float32),
    )(featsN, bf, depthN)


def kernel(x, face_adj, anchors, Wc, bc, Wm, bm):
    bn, cn, fn = x.shape
    fpad = ((fn + ROWS - 1) // ROWS) * ROWS
    while (bn * fpad) % (NC * NS * CHUNK) != 0:
        fpad += ROWS
    n = bn * fpad

    feats = jnp.transpose(x, (0, 2, 1))
    featsP = jnp.pad(feats, ((0, 0), (0, fpad - fn), (0, 0)))
    featsN = featsP.reshape(n, cn)

    Wc4 = jnp.concatenate([Wc[k * cn:(k + 1) * cn] for k in range(4)],
                          axis=1)
    wm2 = Wm.reshape(1, cn)
    bm2 = bm.reshape(1, 1)
    bc2 = bc.reshape(1, cn)

    adjP = jnp.pad(face_adj, ((0, 0), (0, fpad - fn), (0, 0)))
    a0l = adjP[:, :, 0]
    a1l = adjP[:, :, 1]
    a2l = adjP[:, :, 2]
    offs = (jnp.arange(bn, dtype=jnp.int32) * fpad)[:, None]
    a0f = (a0l + offs).reshape(n)
    a1f = (a1l + offs).reshape(n)
    a2f = (a2l + offs).reshape(n)

    qi = jnp.zeros((bn, fpad), jnp.int32).at[:, 0].set(anchors)
    di = jnp.full((bn, fpad), INF, jnp.int32).at[
        jnp.arange(bn), anchors].set(-1)
    tgi = jnp.full((fpad,), -1, jnp.int32)

    z0, z1, z2, z3, init = _k1(featsN, Wc4, wm2, bm2)
    g1, g2, g3 = _k2(z1, z2, z3, a0f, a1f, a2f)
    bf, pred = _k3(z0, g1, g2, g3, bc2, wm2, bm2)
    scores, depth = _k4(a0l, a1l, a2l, pred.reshape(bn, fpad),
                        init.reshape(bn, fpad), qi, di, tgi)
    outfeat = _k5(featsN, bf, depth.reshape(n, 1))

    final_features = outfeat.reshape(bn, fpad, cn)[:, :fn, :]
    final_scores = scores[:, :fn].reshape(bn, fn, 1)
    return final_features, final_scores


# speedup vs baseline: 48.0144x; 1.0521x over previous
import functools

import jax
import jax.numpy as jnp
from jax import lax
from jax.experimental import pallas as pl
from jax.experimental.pallas import tpu as pltpu
from jax.experimental.pallas import tpu_sc as plsc

INF = 2**31 - 1
NC = 2
NS = 16
ROWS = 1000
CHUNK = 128


def _k1_body(feats_ref, wc4_ref, wm_ref, bm_ref, z0_ref, z1_ref, z2_ref,
             z3_ref, init_ref):
    h = feats_ref[...]
    z = jnp.dot(h, wc4_ref[...], preferred_element_type=jnp.float32)
    c = h.shape[1]
    z0_ref[...] = z[:, 0 * c:1 * c]
    z1_ref[...] = z[:, 1 * c:2 * c]
    z2_ref[...] = z[:, 2 * c:3 * c]
    z3_ref[...] = z[:, 3 * c:4 * c]
    s = jnp.sum(h * wm_ref[...], axis=1, keepdims=True) + bm_ref[...]
    init_ref[...] = jax.nn.sigmoid(s)


def _k1(featsN, Wc4, wm2, bm2):
    n, c = featsN.shape
    grid = (n // ROWS,)
    zspec = pl.BlockSpec((ROWS, c), lambda i: (i, 0))
    return pl.pallas_call(
        _k1_body,
        grid=grid,
        in_specs=[
            pl.BlockSpec((ROWS, c), lambda i: (i, 0)),
            pl.BlockSpec((c, 4 * c), lambda i: (0, 0)),
            pl.BlockSpec((1, c), lambda i: (0, 0)),
            pl.BlockSpec((1, 1), lambda i: (0, 0)),
        ],
        out_specs=[zspec, zspec, zspec, zspec,
                   pl.BlockSpec((ROWS, 1), lambda i: (i, 0))],
        out_shape=[jax.ShapeDtypeStruct((n, c), jnp.float32)] * 4
        + [jax.ShapeDtypeStruct((n, 1), jnp.float32)],
    )(featsN, Wc4, wm2, bm2)


def _k2_body(z1, z2, z3, a0, a1, a2, g1, g2, g3, *scr):
    I = scr[0:6]
    R = scr[6:12]
    SI = scr[12:18]
    SG = scr[18:24]
    SO = scr[24:30]
    wid = lax.axis_index("s") * NC + lax.axis_index("c")
    npad = g1.shape[0]
    per_w = npad // (NC * NS)
    nchunk = per_w // CHUNK
    zs = (z1, z2, z3)
    gs = (g1, g2, g3)
    adr = (a0, a1, a2)

    def idx_start(j, s):
        st = wid * per_w + j * CHUNK
        return [pltpu.async_copy(adr[k].at[pl.ds(st, CHUNK)], I[s * 3 + k],
                                 SI[s * 3 + k]) for k in range(3)]

    def gather_start(s):
        return [pltpu.async_copy(zs[k].at[I[s * 3 + k]], R[s * 3 + k],
                                 SG[s * 3 + k]) for k in range(3)]

    def out_start(j, s):
        st = wid * per_w + j * CHUNK
        return [pltpu.async_copy(R[s * 3 + k], gs[k].at[pl.ds(st, CHUNK)],
                                 SO[s * 3 + k]) for k in range(3)]

    idesc = {0: idx_start(0, 0)}
    gdesc = {}
    odesc = {}
    for j in range(nchunk):
        s = j % 2
        for d in idesc[j]:
            d.wait()
        if j >= 2:
            for d in odesc[j - 2]:
                d.wait()
        gdesc[j] = gather_start(s)
        if j >= 1:
            for d in gdesc[j - 1]:
                d.wait()
            odesc[j - 1] = out_start(j - 1, 1 - s)
        if j + 1 < nchunk:
            idesc[j + 1] = idx_start(j + 1, 1 - s)
    for d in gdesc[nchunk - 1]:
        d.wait()
    odesc[nchunk - 1] = out_start(nchunk - 1, (nchunk - 1) % 2)
    for j in (nchunk - 2, nchunk - 1):
        for d in odesc[j]:
            d.wait()


def _k2(z1, z2, z3, a0f, a1f, a2f, npad):
    n, c = z1.shape
    mesh = plsc.VectorSubcoreMesh(core_axis_name="c", subcore_axis_name="s")
    out = jax.ShapeDtypeStruct((npad, c), jnp.float32)
    run = functools.partial(
        pl.kernel,
        out_type=[out, out, out],
        mesh=mesh,
        scratch_types=[pltpu.VMEM((CHUNK,), jnp.int32)] * 6
        + [pltpu.VMEM((CHUNK, c), jnp.float32)] * 6
        + [pltpu.SemaphoreType.DMA] * 18,
    )(_k2_body)
    return run(z1, z2, z3, a0f, a1f, a2f)


def _k3_body(z0_ref, g1_ref, g2_ref, g3_ref, bc_ref, wm_ref, bm_ref,
             bf_ref, pred_ref):
    h = z0_ref[...] + g1_ref[...] + g2_ref[...] + g3_ref[...] + bc_ref[...]
    mu = jnp.mean(h, axis=1, keepdims=True)
    d = h - mu
    var = jnp.mean(d * d, axis=1, keepdims=True)
    bf = d * lax.rsqrt(var + 1e-5)
    bf_ref[...] = bf
    p = jnp.sum(bf * wm_ref[...], axis=1, keepdims=True) + bm_ref[...]
    pred_ref[...] = jax.nn.sigmoid(p)


def _k3(z0, g1, g2, g3, bc2, wm2, bm2):
    n, c = z0.shape
    grid = (n // ROWS,)
    zspec = pl.BlockSpec((ROWS, c), lambda i: (i, 0))
    return pl.pallas_call(
        _k3_body,
        grid=grid,
        in_specs=[zspec, zspec, zspec, zspec,
                  pl.BlockSpec((1, c), lambda i: (0, 0)),
                  pl.BlockSpec((1, c), lambda i: (0, 0)),
                  pl.BlockSpec((1, 1), lambda i: (0, 0))],
        out_specs=[zspec, pl.BlockSpec((ROWS, 1), lambda i: (i, 0))],
        out_shape=[jax.ShapeDtypeStruct((n, c), jnp.float32),
                   jax.ShapeDtypeStruct((n, 1), jnp.float32)],
    )(z0, g1, g2, g3, bc2, wm2, bm2)


def _k4_body(a0h, a1h, a2h, prh, inh, qih, dih, tgh, sch, dph,
             A0, A1, A2, PR, SCR, DQ, QU, TG):
    bn = a0h.shape[0]
    fn = a0h.shape[1]
    wid = lax.axis_index("s") * NC + lax.axis_index("c")
    active = wid < bn
    b = jnp.minimum(wid, bn - 1)

    @pl.when(active)
    def _():
        pltpu.sync_copy(a0h.at[b], A0)
        pltpu.sync_copy(a1h.at[b], A1)
        pltpu.sync_copy(a2h.at[b], A2)
        pltpu.sync_copy(prh.at[b], PR)
        pltpu.sync_copy(inh.at[b], SCR)
        pltpu.sync_copy(qih.at[b], QU.at[pl.ds(0, fn)])
        pltpu.sync_copy(dih.at[b], DQ)
        pltpu.sync_copy(tgh, TG)

    iota16 = lax.iota(jnp.int32, 16)

    def step(carry):
        base, hi, tl, level, tok = carry
        new_lvl = base >= hi
        level = jnp.where(new_lvl, level + 1, level)
        base = jnp.where(new_lvl, hi, base)
        hi = jnp.where(new_lvl, tl, hi)
        lanes = base + iota16
        m = lanes < hi
        fv = plsc.load_gather(QU, [jnp.where(m, lanes, 0)])
        f = jnp.where(m, fv, 0)
        a0v = plsc.load_gather(A0, [f])
        a1v = plsc.load_gather(A1, [f])
        a2v = plsc.load_gather(A2, [f])
        d0 = plsc.load_gather(DQ, [a0v])
        d1 = plsc.load_gather(DQ, [a1v])
        d2 = plsc.load_gather(DQ, [a2v])
        s0 = plsc.load_gather(SCR, [a0v])
        s1 = plsc.load_gather(SCR, [a1v])
        s2 = plsc.load_gather(SCR, [a2v])
        v0 = d0 < level
        v1 = d1 < level
        v2 = d2 < level
        neg = jnp.float32(-1e30)
        nb = jnp.maximum(jnp.maximum(jnp.where(v0, s0, neg),
                                     jnp.where(v1, s1, neg)),
                         jnp.where(v2, s2, neg))
        has = v0 | v1 | v2
        nbv = jnp.where(has, nb, jnp.float32(1.0))
        pf = plsc.load_gather(PR, [f])
        sf = plsc.load_gather(SCR, [f])
        val = jnp.minimum(jnp.maximum(pf, sf), nbv)
        plsc.store_scatter(SCR, [f], val, mask=m)
        dnew = jnp.zeros((16,), jnp.int32) + (level + 1)

        def expand(av, tl, tok):
            dd = plsc.load_gather(DQ, [av])
            cand = m & (dd == INF)
            tokv = tok + iota16
            plsc.store_scatter(TG, [av], tokv, mask=cand)
            tt = plsc.load_gather(TG, [av])
            win = cand & (tt == tokv)
            plsc.store_scatter(DQ, [av], dnew, mask=cand)
            plsc.store_compressed(QU.at[pl.ds(tl, 16)], av, mask=win)
            cnt = plsc.all_reduce_population_count(win)[0]
            return tl + cnt, tok + 16

        tl, tok = expand(a0v, tl, tok)
        tl, tok = expand(a1v, tl, tok)
        tl, tok = expand(a2v, tl, tok)
        return base + 16, hi, tl, level, tok

    def not_done(carry):
        base, hi, tl, _, _ = carry
        return (base < hi) | (hi < tl)

    one_if = jnp.where(active, jnp.int32(1), jnp.int32(0))
    lax.while_loop(not_done, step,
                   (jnp.int32(0), one_if, one_if, jnp.int32(0),
                    jnp.int32(0)))

    @pl.when(active)
    def _():
        pltpu.sync_copy(SCR, sch.at[b])
        pltpu.sync_copy(DQ, dph.at[b])


def _k4(a0l, a1l, a2l, pred2, init2, qi, di, tgi):
    bn, fn = a0l.shape
    mesh = plsc.VectorSubcoreMesh(core_axis_name="c", subcore_axis_name="s")
    run = functools.partial(
        pl.kernel,
        out_type=[jax.ShapeDtypeStruct((bn, fn), jnp.float32),
                  jax.ShapeDtypeStruct((bn, fn), jnp.int32)],
        mesh=mesh,
        scratch_types=[pltpu.VMEM((fn,), jnp.int32)] * 3
        + [pltpu.VMEM((fn,), jnp.float32)] * 2
        + [pltpu.VMEM((fn,), jnp.int32),
           pltpu.VMEM((fn + 16,), jnp.int32),
           pltpu.VMEM((fn,), jnp.int32)],
        compiler_params=pltpu.CompilerParams(needs_layout_passes=False),
    )(_k4_body)
    return run(a0l, a1l, a2l, pred2, init2, qi, di, tgi)


def _k5_body(feats_ref, bf_ref, depth_ref, out_ref):
    reached = depth_ref[...] != INF
    out_ref[...] = jnp.where(reached, bf_ref[...], feats_ref[...])


def _k5(featsN, bf, depthN):
    n, c = featsN.shape
    grid = (n // ROWS,)
    zspec = pl.BlockSpec((ROWS, c), lambda i: (i, 0))
    return pl.pallas_call(
        _k5_body,
        grid=grid,
        in_specs=[zspec, zspec, pl.BlockSpec((ROWS, 1), lambda i: (i, 0))],
        out_specs=zspec,
        out_shape=jax.ShapeDtypeStruct((n, c), jnp.float32),
    )(featsN, bf, depthN)


def kernel(x, face_adj, anchors, Wc, bc, Wm, bm):
    bn, cn, fn = x.shape
    n = bn * fn
    gran = NC * NS * CHUNK
    npad = ((n + gran - 1) // gran) * gran

    feats = jnp.transpose(x, (0, 2, 1))
    featsN = feats.reshape(n, cn)

    Wc4 = jnp.concatenate([Wc[k * cn:(k + 1) * cn] for k in range(4)],
                          axis=1)
    wm2 = Wm.reshape(1, cn)
    bm2 = bm.reshape(1, 1)
    bc2 = bc.reshape(1, cn)

    a0l = face_adj[:, :, 0]
    a1l = face_adj[:, :, 1]
    a2l = face_adj[:, :, 2]
    offs = (jnp.arange(bn, dtype=jnp.int32) * fn)[:, None]
    a0f = jnp.pad((a0l + offs).reshape(n), (0, npad - n))
    a1f = jnp.pad((a1l + offs).reshape(n), (0, npad - n))
    a2f = jnp.pad((a2l + offs).reshape(n), (0, npad - n))

    fq = ((fn + 127) // 128) * 128
    padq = ((0, 0), (0, fq - fn))
    a0q = jnp.pad(a0l, padq)
    a1q = jnp.pad(a1l, padq)
    a2q = jnp.pad(a2l, padq)
    qi = jnp.zeros((bn, fq), jnp.int32).at[:, 0].set(anchors)
    di = jnp.full((bn, fq), INF, jnp.int32).at[
        jnp.arange(bn), anchors].set(-1)
    tgi = jnp.full((fq,), -1, jnp.int32)

    z0, z1, z2, z3, init = _k1(featsN, Wc4, wm2, bm2)
    g1, g2, g3 = _k2(z1, z2, z3, a0f, a1f, a2f, npad)
    bf, pred = _k3(z0, g1, g2, g3, bc2, wm2, bm2)
    predq = jnp.pad(pred.reshape(bn, fn), padq)
    initq = jnp.pad(init.reshape(bn, fn), padq)
    scores, depth = _k4(a0q, a1q, a2q, predq, initq, qi, di, tgi)
    outfeat = _k5(featsN, bf, depth[:, :fn].reshape(n, 1))

    final_features = outfeat.reshape(bn, fn, cn)
    final_scores = scores[:, :fn].reshape(bn, fn, 1)
    return final_features, final_scores
